# baseline (speedup 1.0000x reference)
import functools

import jax
import jax.numpy as jnp
from jax import lax
from jax.experimental import pallas as pl
from jax.experimental.pallas import tpu as pltpu
from jax.experimental.pallas import tpu_sc as plsc

_K = 8192
_D = 256
_R = 16384
_BM = 2048
_BK = 512
_NM = _R // _BM
_NK = _K // _BK

_NW = 32
_TPW = _R // _NW
_GCH = 128


_WIN = 2736
_INT_MAX = 2**31 - 1


def _bf16_rne(x):
    u = lax.bitcast_convert_type(x, jnp.uint32)
    r = (u + jnp.uint32(0x7FFF) + ((u >> 16) & jnp.uint32(1))) \
        & jnp.uint32(0xFFFF0000)
    return lax.bitcast_convert_type(r, jnp.float32)


def _dist_argmin_body(a_ref, r_ref, t_ref, c_ref, idx_ref, wv, wi, gv, gi):
    k = pl.program_id(1)

    @pl.when(k == 0)
    def _init():
        wv[...] = jnp.full_like(wv, jnp.inf)
        wi[...] = jnp.zeros_like(wi)
        gv[...] = jnp.full_like(gv, jnp.inf)
        gi[...] = jnp.zeros_like(gi)

    m = lax.dot_general(r_ref[...].astype(jnp.bfloat16),
                        t_ref[...].astype(jnp.bfloat16),
                        (((1,), (1,)), ((), ())),
                        preferred_element_type=jnp.float32)
    dist = (a_ref[...] - 2.0 * m) + c_ref[...]
    lcol = lax.broadcasted_iota(jnp.int32, (_BM, _BK), 1)
    iota = lcol + k * _BK

    def fold(mask):
        d = jnp.where(mask, dist, jnp.inf) if mask is not None else dist
        rowmin = jnp.min(d, axis=1, keepdims=True)
        cand = jnp.min(jnp.where(d == rowmin, iota, jnp.int32(_INT_MAX)),
                       axis=1, keepdims=True)
        upd = rowmin < wv[...]
        wi[...] = jnp.where(upd, cand, wi[...])
        wv[...] = jnp.where(upd, rowmin, wv[...])

    def merge():
        upd = wv[...] < gv[...]
        gi[...] = jnp.where(upd, wi[...], gi[...])
        gv[...] = _bf16_rne(jnp.where(upd, wv[...], gv[...]))
        wv[...] = jnp.full_like(wv, jnp.inf)
        wi[...] = jnp.zeros_like(wi)

    for b in range(1, (_K + _WIN - 1) // _WIN):
        blk, col = (b * _WIN) // _BK, (b * _WIN) % _BK

        @pl.when(k == blk)
        def _split(col=col):
            fold(lcol < col)
            merge()
            fold(lcol >= col)

    bset = {(b * _WIN) // _BK for b in range(1, (_K + _WIN - 1) // _WIN)}
    cond = True
    for blk in bset:
        cond = cond & (k != blk)

    @pl.when(cond)
    def _full():
        fold(None)

    @pl.when(k == _NK - 1)
    def _out():
        upd = wv[...] < gv[...]
        idx_ref[...] = jnp.where(upd, wi[...], gi[...])


_dist_argmin_call = pl.pallas_call(
    _dist_argmin_body,
    grid=(_NM, _NK),
    in_specs=[
        pl.BlockSpec((_BM, 1), lambda m, k: (m, 0)),
        pl.BlockSpec((_BM, _D), lambda m, k: (m, 0)),
        pl.BlockSpec((_BK, _D), lambda m, k: (k, 0)),
        pl.BlockSpec((1, _BK), lambda m, k: (0, k)),
    ],
    out_specs=pl.BlockSpec((_BM, 1), lambda m, k: (m, 0)),
    out_shape=jax.ShapeDtypeStruct((_R, 1), jnp.int32),
    scratch_shapes=[
        pltpu.VMEM((_BM, 1), jnp.float32),
        pltpu.VMEM((_BM, 1), jnp.int32),
        pltpu.VMEM((_BM, 1), jnp.float32),
        pltpu.VMEM((_BM, 1), jnp.int32),
    ],
)


def _update_body(r_ref, zq_ref, acc_ref, rn_ref, accn_ref, dsq_ref, dacc):
    mblk = pl.program_id(0)

    @pl.when(mblk == 0)
    def _init():
        dacc[...] = jnp.zeros_like(dacc)

    r = r_ref[...]
    zq = zq_ref[...]
    d = zq - r
    zq_st = r + d
    rn_ref[...] = r - zq_st
    accn_ref[...] = acc_ref[...] + zq_st
    dacc[...] = dacc[...] + jnp.sum(d * d)

    @pl.when(mblk == _NM - 1)
    def _out():
        dsq_ref[...] = dacc[...]


_update_call = pl.pallas_call(
    _update_body,
    grid=(_NM,),
    in_specs=[
        pl.BlockSpec((_BM, _D), lambda m: (m, 0)),
        pl.BlockSpec((_BM, _D), lambda m: (m, 0)),
        pl.BlockSpec((_BM, _D), lambda m: (m, 0)),
    ],
    out_specs=[
        pl.BlockSpec((_BM, _D), lambda m: (m, 0)),
        pl.BlockSpec((_BM, _D), lambda m: (m, 0)),
        pl.BlockSpec((1, 1), lambda m: (0, 0)),
    ],
    out_shape=[
        jax.ShapeDtypeStruct((_R, _D), jnp.float32),
        jax.ShapeDtypeStruct((_R, _D), jnp.float32),
        jax.ShapeDtypeStruct((1, 1), jnp.float32),
    ],
    scratch_shapes=[pltpu.VMEM((1, 1), jnp.float32)],
)


def _mlp_loss_body(zq_ref, tgt_ref, w1_ref, b1_ref, w2_ref, b2_ref,
                   out_ref, acc):
    mblk = pl.program_id(0)

    @pl.when(mblk == 0)
    def _init():
        acc[...] = jnp.zeros_like(acc)

    h = jnp.dot(zq_ref[...], w1_ref[...],
                preferred_element_type=jnp.float32) + b1_ref[...]
    g = 0.5 * h * (1.0 + lax.erf(h * (1.0 / jnp.sqrt(2.0).astype(jnp.float32))))
    p = jnp.dot(g, w2_ref[...],
                preferred_element_type=jnp.float32) + b2_ref[...]
    e = p - tgt_ref[...]
    acc[...] = acc[...] + jnp.sum(e * e)

    @pl.when(mblk == _NM - 1)
    def _out():
        out_ref[...] = acc[...]


_mlp_loss_call = pl.pallas_call(
    _mlp_loss_body,
    grid=(_NM,),
    in_specs=[
        pl.BlockSpec((_BM, _D), lambda m: (m, 0)),
        pl.BlockSpec((_BM, _D), lambda m: (m, 0)),
        pl.BlockSpec((_D, _D), lambda m: (0, 0)),
        pl.BlockSpec((1, _D), lambda m: (0, 0)),
        pl.BlockSpec((_D, _D), lambda m: (0, 0)),
        pl.BlockSpec((1, _D), lambda m: (0, 0)),
    ],
    out_specs=pl.BlockSpec((1, 1), lambda m: (0, 0)),
    out_shape=jax.ShapeDtypeStruct((1, 1), jnp.float32),
    scratch_shapes=[pltpu.VMEM((1, 1), jnp.float32)],
)


def _sc_gather_body(table_hbm, idx_hbm, out_hbm, idx_v, rows_v, sem):
    wid = lax.axis_index("s") * 2 + lax.axis_index("c")
    base = wid * _TPW
    for c in range(_TPW // _GCH):
        pltpu.sync_copy(idx_hbm.at[pl.ds(base + c * _GCH, _GCH)], idx_v)
        pltpu.async_copy(table_hbm.at[idx_v], rows_v, sem).wait()
        pltpu.sync_copy(rows_v, out_hbm.at[pl.ds(base + c * _GCH, _GCH)])


@functools.cache
def _sc_gather_call():
    return pl.kernel(
        _sc_gather_body,
        mesh=plsc.VectorSubcoreMesh(core_axis_name="c", subcore_axis_name="s",
                                    num_cores=2),
        out_type=jax.ShapeDtypeStruct((_R, _D), jnp.float32),
        scratch_types=[
            pltpu.VMEM((_GCH,), jnp.int32),
            pltpu.VMEM((_GCH, _D), jnp.float32),
            pltpu.SemaphoreType.DMA,
        ],
    )


def kernel(z, w2v_targets, tables, W1, b1, W2, b2):
    Bz, Tz, Dz = z.shape
    flat = z.reshape(_R, _D)
    tgt = w2v_targets.reshape(_R, -1)
    tsq = jnp.sum(tables ** 2, axis=-1)

    r = flat
    acc = jnp.zeros_like(flat)
    codes = []
    dsqs = []
    zq_st0 = None
    for i in range(tables.shape[0]):
        a = jnp.sum(r ** 2, axis=1, keepdims=True)
        idx2d = _dist_argmin_call(a, r, tables[i], tsq[i][None, :])
        zq = _sc_gather_call()(tables[i], idx2d.reshape(_R))
        r, acc, dsq = _update_call(r, zq, acc)
        codes.append(idx2d.reshape(Bz, Tz))
        dsqs.append(dsq[0, 0])
        if i == 0:
            zq_st0 = acc

    n_el = float(_R * _D)
    total_vq_loss = jnp.float32(0.0)
    for dsq in dsqs:
        emb = dsq / n_el
        total_vq_loss = total_vq_loss + (emb + jnp.float32(0.25) * emb)

    sem_sum = _mlp_loss_call(zq_st0, tgt, W1, b1[None, :], W2, b2[None, :])
    semantic_loss = sem_sum[0, 0] / jnp.float32(_R * tgt.shape[1])

    z_q_total = acc.reshape(Bz, Tz, Dz)
    all_codes = jnp.stack(codes, axis=-1)
    return (z_q_total, all_codes, all_codes[..., 0], total_vq_loss,
            semantic_loss)
